# baseline (speedup 1.0000x reference)
import functools
import math

import jax
import jax.numpy as jnp
from jax import lax
from jax.experimental import pallas as pl
from jax.experimental.pallas import tpu as pltpu
from jax.experimental.pallas import tpu_sc as plsc

N_NODES = 10000
D = 128
NGRAPH = 64
NCLS = 10

NC = 2
NS = 16
NW = NC * NS
CHUNK = 128
NBUF = 2
CHUNKS_PER_TILE = 80
IDX_GROUP = 16
EDGES_PER_TILE = CHUNKS_PER_TILE * CHUNK
E_PAD = NW * EDGES_PER_TILE
ACC_ROWS = 10240
ROWS_PER_TILE = ACC_ROWS // NS
DUMMY = N_NODES

_sc_mesh = plsc.VectorSubcoreMesh(core_axis_name="c", subcore_axis_name="s")


@functools.partial(
    pl.kernel,
    out_type=jax.ShapeDtypeStruct((NW, ACC_ROWS), jnp.float32),
    mesh=_sc_mesh,
    scratch_types=[
        pltpu.VMEM((EDGES_PER_TILE,), jnp.int32),
        pltpu.VMEM((ACC_ROWS,), jnp.float32),
    ],
    compiler_params=pltpu.CompilerParams(needs_layout_passes=False),
)
def _sc_degree(col_hbm, out_hbm, col_v, cnt_v):
    c = lax.axis_index("c")
    s = lax.axis_index("s")
    wid = s * NC + c
    pltpu.sync_copy(
        col_hbm.at[pl.ds(wid * EDGES_PER_TILE, EDGES_PER_TILE)], col_v)
    zeros = jnp.zeros((16,), jnp.float32)

    def zbody(i, carry):
        cnt_v[pl.ds(i * 16, 16)] = zeros
        return carry

    lax.fori_loop(0, ACC_ROWS // 16, zbody, 0)
    ones = jnp.ones((16,), jnp.float32)

    def body(i, carry):
        idx = col_v[pl.ds(i * 16, 16)]
        plsc.addupdate_scatter(cnt_v, [idx], ones)
        return carry

    lax.fori_loop(0, EDGES_PER_TILE // 16, body, 0)
    pltpu.sync_copy(cnt_v, out_hbm.at[wid])


def _tc_scale_body(x_ref, w_ref, cnt_ref, xs_ref, dinv_ref):
    cnt = jnp.sum(cnt_ref[...], axis=0)[:N_NODES]
    dinv = lax.rsqrt(cnt + 1.0)
    x = jnp.dot(x_ref[...], w_ref[...], preferred_element_type=jnp.float32)
    xs_ref[...] = x * dinv[:, None]
    dinv_ref[...] = dinv


_tc_scale = pl.pallas_call(
    _tc_scale_body,
    out_shape=(
        jax.ShapeDtypeStruct((N_NODES, D), jnp.float32),
        jax.ShapeDtypeStruct((N_NODES,), jnp.float32),
    ),
)


@functools.partial(
    pl.kernel,
    out_type=jax.ShapeDtypeStruct((NC, ACC_ROWS, D), jnp.float32),
    mesh=_sc_mesh,
    scratch_types=[
        pltpu.VMEM((IDX_GROUP, CHUNK), jnp.int32),
        pltpu.VMEM((IDX_GROUP, CHUNK), jnp.int32),
        pltpu.VMEM((CHUNK, D), jnp.float32),
        pltpu.VMEM((CHUNK, D), jnp.float32),
        pltpu.VMEM_SHARED((ACC_ROWS, D), jnp.float32),
        pltpu.SemaphoreType.DMA,
        pltpu.SemaphoreType.DMA,
        pltpu.SemaphoreType.DMA,
        pltpu.SemaphoreType.DMA,
    ],
)
def _sc_edges(row_hbm, col_hbm, xs_hbm, out_hbm, row_v, col_v,
              b0, b1, accum, g0, g1, s0, s1):
    c = lax.axis_index("c")
    s = lax.axis_index("s")
    wid = s * NC + c
    bufs = (b0, b1)
    gsem = (g0, g1)
    ssem = (s0, s1)

    zeros = jnp.zeros((16,), jnp.float32)

    def zbody(i, carry):
        b0[i // (D // 16), pl.ds((i % (D // 16)) * 16, 16)] = zeros
        return carry

    lax.fori_loop(0, CHUNK * D // 16, zbody, 0)
    for r in range(ROWS_PER_TILE // CHUNK):
        pltpu.sync_copy(
            b0, accum.at[pl.ds(s * ROWS_PER_TILE + r * CHUNK, CHUNK)])
    plsc.subcore_barrier()

    for grp in range(CHUNKS_PER_TILE // IDX_GROUP):
        pltpu.sync_copy(row_hbm.at[wid, pl.ds(grp * IDX_GROUP, IDX_GROUP)],
                        row_v)
        pltpu.sync_copy(col_hbm.at[wid, pl.ds(grp * IDX_GROUP, IDX_GROUP)],
                        col_v)
        for k in range(NBUF):
            pltpu.async_copy(xs_hbm.at[row_v.at[k]], bufs[k], gsem[k])

        def pbody(it, carry):
            base = it * NBUF
            for k in range(NBUF):
                pltpu.make_async_copy(
                    xs_hbm.at[row_v.at[base + k]], bufs[k], gsem[k]).wait()
                pltpu.async_copy(bufs[k], accum.at[col_v.at[base + k]],
                                 ssem[k], add=True)
            nxt = base + NBUF
            for k in range(NBUF):
                pltpu.make_async_copy(
                    xs_hbm.at[row_v.at[base + k]], bufs[k], ssem[k]).wait()
                pltpu.async_copy(
                    xs_hbm.at[row_v.at[nxt + k]], bufs[k], gsem[k])
            return carry

        lax.fori_loop(0, IDX_GROUP // NBUF - 1, pbody, 0)
        base = IDX_GROUP - NBUF
        for k in range(NBUF):
            pltpu.make_async_copy(
                xs_hbm.at[row_v.at[base + k]], bufs[k], gsem[k]).wait()
            pltpu.async_copy(bufs[k], accum.at[col_v.at[base + k]],
                             ssem[k], add=True)
        for k in range(NBUF):
            pltpu.make_async_copy(
                xs_hbm.at[row_v.at[base + k]], bufs[k], ssem[k]).wait()
    plsc.subcore_barrier()

    for r in range(ROWS_PER_TILE // CHUNK):
        off = s * ROWS_PER_TILE + r * CHUNK
        pltpu.sync_copy(accum.at[pl.ds(off, CHUNK)], b0)
        pltpu.sync_copy(b0, out_hbm.at[c, pl.ds(off, CHUNK)])


def _tc_final_body(p_ref, xs_ref, dinv_ref, bg_ref, bidx_ref, wfc_ref,
                   bfc_ref, gam_ref, bet_ref, hid_ref, score_ref):
    total = p_ref[0, :N_NODES, :] + p_ref[1, :N_NODES, :] + xs_ref[...]
    hidden = total * dinv_ref[...][:, None] + bg_ref[...][None, :]
    hid_ref[...] = hidden
    gids = lax.broadcasted_iota(jnp.int32, (NGRAPH, N_NODES), 0)
    onehot = (gids == bidx_ref[...][None, :]).astype(jnp.float32)
    sums = jnp.dot(onehot, hidden, preferred_element_type=jnp.float32)
    counts = jnp.sum(onehot, axis=1)
    pooled = sums / jnp.maximum(counts, 1.0)[:, None]
    logits = (jnp.dot(pooled, wfc_ref[...], preferred_element_type=jnp.float32)
              + bfc_ref[...][None, :])
    score_ref[...] = (logits * (1.0 / math.sqrt(1.0 + 1e-5))
                      * gam_ref[...][None, :] + bet_ref[...][None, :])


_tc_final = pl.pallas_call(
    _tc_final_body,
    out_shape=(
        jax.ShapeDtypeStruct((N_NODES, D), jnp.float32),
        jax.ShapeDtypeStruct((NGRAPH, NCLS), jnp.float32),
    ),
)


def kernel(input_rep, batch_idx, graph_size, edge_index, W_gcn, b_gcn,
           W_fc, b_fc, bn_gamma, bn_beta):
    del graph_size
    row = edge_index[0].astype(jnp.int32)
    col = edge_index[1].astype(jnp.int32)
    n_edges = row.shape[0]
    pad = E_PAD - n_edges
    pad_iota = jnp.arange(pad, dtype=jnp.int32)
    rows_p = jnp.concatenate([row, pad_iota % N_NODES])
    cols_p = jnp.concatenate([col, DUMMY + pad_iota % (ACC_ROWS - N_NODES)])
    rows3 = rows_p.reshape(NW, CHUNKS_PER_TILE, CHUNK)
    cols3 = cols_p.reshape(NW, CHUNKS_PER_TILE, CHUNK)

    cnt = _sc_degree(cols_p)
    xs, dinv = _tc_scale(input_rep, W_gcn, cnt)
    partials = _sc_edges(rows3, cols3, xs)
    hidden, score = _tc_final(partials, xs, dinv, b_gcn,
                              batch_idx.astype(jnp.int32), W_fc, b_fc,
                              bn_gamma, bn_beta)
    return (hidden, score)


# speedup vs baseline: 1.1628x; 1.1628x over previous
import functools
import math

import jax
import jax.numpy as jnp
from jax import lax
from jax.experimental import pallas as pl
from jax.experimental.pallas import tpu as pltpu
from jax.experimental.pallas import tpu_sc as plsc

N_NODES = 10000
D = 128
NGRAPH = 64
NCLS = 10

NC = 2
NS = 16
NW = NC * NS
CHUNK = 64
NBUF = 4
CHUNKS_PER_TILE = 160
IDX_GROUP = 40
EDGES_PER_TILE = CHUNKS_PER_TILE * CHUNK
E_PAD = NW * EDGES_PER_TILE
ACC_ROWS = 10240
ROWS_PER_TILE = ACC_ROWS // NS
DUMMY = N_NODES

_sc_mesh = plsc.VectorSubcoreMesh(core_axis_name="c", subcore_axis_name="s")


@functools.partial(
    pl.kernel,
    out_type=jax.ShapeDtypeStruct((NW, ACC_ROWS), jnp.float32),
    mesh=_sc_mesh,
    scratch_types=[
        pltpu.VMEM((EDGES_PER_TILE,), jnp.int32),
        pltpu.VMEM((ACC_ROWS,), jnp.float32),
    ],
    compiler_params=pltpu.CompilerParams(needs_layout_passes=False),
)
def _sc_degree(col_hbm, out_hbm, col_v, cnt_v):
    c = lax.axis_index("c")
    s = lax.axis_index("s")
    wid = s * NC + c
    pltpu.sync_copy(
        col_hbm.at[pl.ds(wid * EDGES_PER_TILE, EDGES_PER_TILE)], col_v)
    zeros = jnp.zeros((16,), jnp.float32)

    def zbody(i, carry):
        cnt_v[pl.ds(i * 16, 16)] = zeros
        return carry

    lax.fori_loop(0, ACC_ROWS // 16, zbody, 0)
    ones = jnp.ones((16,), jnp.float32)

    def body(i, carry):
        idx = col_v[pl.ds(i * 16, 16)]
        plsc.addupdate_scatter(cnt_v, [idx], ones)
        return carry

    lax.fori_loop(0, EDGES_PER_TILE // 16, body, 0)
    pltpu.sync_copy(cnt_v, out_hbm.at[wid])


def _tc_matmul_body(x_ref, w_ref, o_ref):
    o_ref[...] = jnp.dot(x_ref[...], w_ref[...],
                         preferred_element_type=jnp.float32)


_tc_matmul = pl.pallas_call(
    _tc_matmul_body,
    out_shape=jax.ShapeDtypeStruct((N_NODES, D), jnp.float32),
)


def _tc_scale_body(x_ref, cnt_ref, xs_ref, dinv_ref):
    cnt = jnp.sum(cnt_ref[...], axis=0)[:N_NODES]
    dinv = lax.rsqrt(cnt + 1.0)
    xs_ref[...] = x_ref[...] * dinv[:, None]
    dinv_ref[...] = dinv


_tc_scale = pl.pallas_call(
    _tc_scale_body,
    out_shape=(
        jax.ShapeDtypeStruct((N_NODES, D), jnp.float32),
        jax.ShapeDtypeStruct((N_NODES,), jnp.float32),
    ),
)


@functools.partial(
    pl.kernel,
    out_type=jax.ShapeDtypeStruct((NC, ACC_ROWS, D), jnp.float32),
    mesh=_sc_mesh,
    scratch_types=[
        pltpu.VMEM((IDX_GROUP, CHUNK), jnp.int32),
        pltpu.VMEM((IDX_GROUP, CHUNK), jnp.int32),
        pltpu.VMEM((CHUNK, D), jnp.float32),
        pltpu.VMEM((CHUNK, D), jnp.float32),
        pltpu.VMEM((CHUNK, D), jnp.float32),
        pltpu.VMEM((CHUNK, D), jnp.float32),
        pltpu.VMEM_SHARED((ACC_ROWS, D), jnp.float32),
        pltpu.SemaphoreType.DMA,
        pltpu.SemaphoreType.DMA,
        pltpu.SemaphoreType.DMA,
        pltpu.SemaphoreType.DMA,
        pltpu.SemaphoreType.DMA,
        pltpu.SemaphoreType.DMA,
        pltpu.SemaphoreType.DMA,
        pltpu.SemaphoreType.DMA,
    ],
)
def _sc_edges(row_hbm, col_hbm, xs_hbm, out_hbm, row_v, col_v,
              b0, b1, b2, b3, accum, g0, g1, g2, g3, s0, s1, s2, s3):
    c = lax.axis_index("c")
    s = lax.axis_index("s")
    wid = s * NC + c
    bufs = (b0, b1, b2, b3)
    gsem = (g0, g1, g2, g3)
    ssem = (s0, s1, s2, s3)

    zeros = jnp.zeros((16,), jnp.float32)

    def zbody(i, carry):
        b0[i // (D // 16), pl.ds((i % (D // 16)) * 16, 16)] = zeros
        return carry

    lax.fori_loop(0, CHUNK * D // 16, zbody, 0)
    for r in range(ROWS_PER_TILE // CHUNK):
        pltpu.sync_copy(
            b0, accum.at[pl.ds(s * ROWS_PER_TILE + r * CHUNK, CHUNK)])
    plsc.subcore_barrier()

    for grp in range(CHUNKS_PER_TILE // IDX_GROUP):
        pltpu.sync_copy(row_hbm.at[wid, pl.ds(grp * IDX_GROUP, IDX_GROUP)],
                        row_v)
        pltpu.sync_copy(col_hbm.at[wid, pl.ds(grp * IDX_GROUP, IDX_GROUP)],
                        col_v)
        for k in range(NBUF):
            pltpu.async_copy(xs_hbm.at[row_v.at[k]], bufs[k], gsem[k])

        def pbody(it, carry):
            base = it * NBUF
            for k in range(NBUF):
                pltpu.make_async_copy(
                    xs_hbm.at[row_v.at[base + k]], bufs[k], gsem[k]).wait()
                pltpu.async_copy(bufs[k], accum.at[col_v.at[base + k]],
                                 ssem[k], add=True)
            nxt = base + NBUF
            for k in range(NBUF):
                pltpu.make_async_copy(
                    xs_hbm.at[row_v.at[base + k]], bufs[k], ssem[k]).wait()
                pltpu.async_copy(
                    xs_hbm.at[row_v.at[nxt + k]], bufs[k], gsem[k])
            return carry

        lax.fori_loop(0, IDX_GROUP // NBUF - 1, pbody, 0)
        base = IDX_GROUP - NBUF
        for k in range(NBUF):
            pltpu.make_async_copy(
                xs_hbm.at[row_v.at[base + k]], bufs[k], gsem[k]).wait()
            pltpu.async_copy(bufs[k], accum.at[col_v.at[base + k]],
                             ssem[k], add=True)
        for k in range(NBUF):
            pltpu.make_async_copy(
                xs_hbm.at[row_v.at[base + k]], bufs[k], ssem[k]).wait()
    plsc.subcore_barrier()

    for r in range(ROWS_PER_TILE // CHUNK):
        off = s * ROWS_PER_TILE + r * CHUNK
        pltpu.sync_copy(accum.at[pl.ds(off, CHUNK)], b0)
        pltpu.sync_copy(b0, out_hbm.at[c, pl.ds(off, CHUNK)])


def _tc_final_body(p_ref, xs_ref, dinv_ref, bg_ref, bidx_ref, wfc_ref,
                   bfc_ref, gam_ref, bet_ref, hid_ref, score_ref):
    total = p_ref[0, :N_NODES, :] + p_ref[1, :N_NODES, :] + xs_ref[...]
    hidden = total * dinv_ref[...][:, None] + bg_ref[...][None, :]
    hid_ref[...] = hidden
    gids = lax.broadcasted_iota(jnp.int32, (NGRAPH, N_NODES), 0)
    onehot = (gids == bidx_ref[...][None, :]).astype(jnp.float32)
    sums = jnp.dot(onehot, hidden, preferred_element_type=jnp.float32)
    counts = jnp.sum(onehot, axis=1)
    pooled = sums / jnp.maximum(counts, 1.0)[:, None]
    logits = (jnp.dot(pooled, wfc_ref[...], preferred_element_type=jnp.float32)
              + bfc_ref[...][None, :])
    score_ref[...] = (logits * (1.0 / math.sqrt(1.0 + 1e-5))
                      * gam_ref[...][None, :] + bet_ref[...][None, :])


_tc_final = pl.pallas_call(
    _tc_final_body,
    out_shape=(
        jax.ShapeDtypeStruct((N_NODES, D), jnp.float32),
        jax.ShapeDtypeStruct((NGRAPH, NCLS), jnp.float32),
    ),
)


def kernel(input_rep, batch_idx, graph_size, edge_index, W_gcn, b_gcn,
           W_fc, b_fc, bn_gamma, bn_beta):
    del graph_size
    row = edge_index[0].astype(jnp.int32)
    col = edge_index[1].astype(jnp.int32)
    n_edges = row.shape[0]
    pad = E_PAD - n_edges
    pad_iota = jnp.arange(pad, dtype=jnp.int32)
    rows_p = jnp.concatenate([row, pad_iota % N_NODES])
    cols_p = jnp.concatenate([col, DUMMY + pad_iota % (ACC_ROWS - N_NODES)])
    rows3 = rows_p.reshape(NW, CHUNKS_PER_TILE, CHUNK)
    cols3 = cols_p.reshape(NW, CHUNKS_PER_TILE, CHUNK)

    x_raw = _tc_matmul(input_rep, W_gcn)
    cnt = _sc_degree(cols_p)
    xs, dinv = _tc_scale(x_raw, cnt)
    partials = _sc_edges(rows3, cols3, xs)
    hidden, score = _tc_final(partials, xs, dinv, b_gcn,
                              batch_idx.astype(jnp.int32), W_fc, b_fc,
                              bn_gamma, bn_beta)
    return (hidden, score)


# speedup vs baseline: 1.1750x; 1.0105x over previous
import functools
import math

import jax
import jax.numpy as jnp
from jax import lax
from jax.experimental import pallas as pl
from jax.experimental.pallas import tpu as pltpu
from jax.experimental.pallas import tpu_sc as plsc

N_NODES = 10000
D = 128
NGRAPH = 64
NCLS = 10

NC = 2
NS = 16
NW = NC * NS
CHUNK = 64
NBUF = 4
CHUNKS_PER_TILE = 160
IDX_GROUP = 40
EDGES_PER_TILE = CHUNKS_PER_TILE * CHUNK
E_PAD = NW * EDGES_PER_TILE
ACC_ROWS = 10240
ROWS_PER_TILE = ACC_ROWS // NS
DUMMY = N_NODES

_sc_mesh = plsc.VectorSubcoreMesh(core_axis_name="c", subcore_axis_name="s")


@functools.partial(
    pl.kernel,
    out_type=jax.ShapeDtypeStruct((NW, ACC_ROWS), jnp.float32),
    mesh=_sc_mesh,
    scratch_types=[
        pltpu.VMEM((EDGES_PER_TILE,), jnp.int32),
        pltpu.VMEM((ACC_ROWS,), jnp.float32),
    ],
    compiler_params=pltpu.CompilerParams(needs_layout_passes=False),
)
def _sc_degree(col_hbm, out_hbm, col_v, cnt_v):
    c = lax.axis_index("c")
    s = lax.axis_index("s")
    wid = s * NC + c
    pltpu.sync_copy(
        col_hbm.at[pl.ds(wid * EDGES_PER_TILE, EDGES_PER_TILE)], col_v)
    zeros = jnp.zeros((16,), jnp.float32)

    def zbody(i, carry):
        cnt_v[pl.ds(i * 16, 16)] = zeros
        return carry

    lax.fori_loop(0, ACC_ROWS // 16, zbody, 0)
    ones = jnp.ones((16,), jnp.float32)

    def body(i, carry):
        idx = col_v[pl.ds(i * 16, 16)]
        plsc.addupdate_scatter(cnt_v, [idx], ones)
        return carry

    lax.fori_loop(0, EDGES_PER_TILE // 16, body, 0)
    pltpu.sync_copy(cnt_v, out_hbm.at[wid])


def _tc_scale_body(x_ref, w_ref, cnt_ref, xs_ref, dinv_ref):
    cnt = jnp.sum(cnt_ref[...], axis=0)[:N_NODES]
    dinv = lax.rsqrt(cnt + 1.0)
    x = jnp.dot(x_ref[...], w_ref[...], preferred_element_type=jnp.float32)
    xs_ref[...] = x * dinv[:, None]
    dinv_ref[...] = dinv


_tc_scale = pl.pallas_call(
    _tc_scale_body,
    out_shape=(
        jax.ShapeDtypeStruct((N_NODES, D), jnp.float32),
        jax.ShapeDtypeStruct((N_NODES,), jnp.float32),
    ),
)


@functools.partial(
    pl.kernel,
    out_type=jax.ShapeDtypeStruct((NC, ACC_ROWS, D), jnp.float32),
    mesh=_sc_mesh,
    scratch_types=(
        [pltpu.VMEM((IDX_GROUP, CHUNK), jnp.int32)] * 2
        + [pltpu.VMEM((CHUNK, D), jnp.float32)] * NBUF
        + [pltpu.VMEM_SHARED((ACC_ROWS, D), jnp.float32)]
        + [pltpu.SemaphoreType.DMA] * (2 * NBUF)
    ),
)
def _sc_edges(row_hbm, col_hbm, xs_hbm, out_hbm, row_v, col_v, *scr):
    c = lax.axis_index("c")
    s = lax.axis_index("s")
    wid = s * NC + c
    bufs = scr[:NBUF]
    accum = scr[NBUF]
    gsem = scr[NBUF + 1:2 * NBUF + 1]
    ssem = scr[2 * NBUF + 1:]
    b0 = bufs[0]

    zeros = jnp.zeros((16,), jnp.float32)

    def zbody(i, carry):
        b0[i // (D // 16), pl.ds((i % (D // 16)) * 16, 16)] = zeros
        return carry

    lax.fori_loop(0, CHUNK * D // 16, zbody, 0)
    for r in range(ROWS_PER_TILE // CHUNK):
        pltpu.sync_copy(
            b0, accum.at[pl.ds(s * ROWS_PER_TILE + r * CHUNK, CHUNK)])
    plsc.subcore_barrier()

    for grp in range(CHUNKS_PER_TILE // IDX_GROUP):
        pltpu.sync_copy(row_hbm.at[wid, pl.ds(grp * IDX_GROUP, IDX_GROUP)],
                        row_v)
        pltpu.sync_copy(col_hbm.at[wid, pl.ds(grp * IDX_GROUP, IDX_GROUP)],
                        col_v)
        for k in range(NBUF):
            pltpu.async_copy(xs_hbm.at[row_v.at[k]], bufs[k], gsem[k])

        def pbody(it, carry):
            base = it * NBUF
            for k in range(NBUF):
                pltpu.make_async_copy(
                    xs_hbm.at[row_v.at[base + k]], bufs[k], gsem[k]).wait()
                pltpu.async_copy(bufs[k], accum.at[col_v.at[base + k]],
                                 ssem[k], add=True)
            nxt = base + NBUF
            for k in range(NBUF):
                pltpu.make_async_copy(
                    xs_hbm.at[row_v.at[base + k]], bufs[k], ssem[k]).wait()
                pltpu.async_copy(
                    xs_hbm.at[row_v.at[nxt + k]], bufs[k], gsem[k])
            return carry

        lax.fori_loop(0, IDX_GROUP // NBUF - 1, pbody, 0)
        base = IDX_GROUP - NBUF
        for k in range(NBUF):
            pltpu.make_async_copy(
                xs_hbm.at[row_v.at[base + k]], bufs[k], gsem[k]).wait()
            pltpu.async_copy(bufs[k], accum.at[col_v.at[base + k]],
                             ssem[k], add=True)
        for k in range(NBUF):
            pltpu.make_async_copy(
                xs_hbm.at[row_v.at[base + k]], bufs[k], ssem[k]).wait()
    plsc.subcore_barrier()

    pltpu.sync_copy(accum.at[pl.ds(s * ROWS_PER_TILE, ROWS_PER_TILE)],
                    out_hbm.at[c, pl.ds(s * ROWS_PER_TILE, ROWS_PER_TILE)])


def _tc_final_body(p_ref, xs_ref, dinv_ref, bg_ref, bidx_ref, wfc_ref,
                   bfc_ref, gam_ref, bet_ref, hid_ref, score_ref):
    total = p_ref[0, :N_NODES, :] + p_ref[1, :N_NODES, :] + xs_ref[...]
    hidden = total * dinv_ref[...][:, None] + bg_ref[...][None, :]
    hid_ref[...] = hidden
    gids = lax.broadcasted_iota(jnp.int32, (NGRAPH, N_NODES), 0)
    onehot = (gids == bidx_ref[...][None, :]).astype(jnp.float32)
    sums = jnp.dot(onehot, hidden, preferred_element_type=jnp.float32)
    counts = jnp.sum(onehot, axis=1)
    pooled = sums / jnp.maximum(counts, 1.0)[:, None]
    logits = (jnp.dot(pooled, wfc_ref[...], preferred_element_type=jnp.float32)
              + bfc_ref[...][None, :])
    score_ref[...] = (logits * (1.0 / math.sqrt(1.0 + 1e-5))
                      * gam_ref[...][None, :] + bet_ref[...][None, :])


_tc_final = pl.pallas_call(
    _tc_final_body,
    out_shape=(
        jax.ShapeDtypeStruct((N_NODES, D), jnp.float32),
        jax.ShapeDtypeStruct((NGRAPH, NCLS), jnp.float32),
    ),
)


def kernel(input_rep, batch_idx, graph_size, edge_index, W_gcn, b_gcn,
           W_fc, b_fc, bn_gamma, bn_beta):
    del graph_size
    row = edge_index[0].astype(jnp.int32)
    col = edge_index[1].astype(jnp.int32)
    n_edges = row.shape[0]
    pad = E_PAD - n_edges
    pad_iota = jnp.arange(pad, dtype=jnp.int32)
    rows_p = jnp.concatenate([row, pad_iota % N_NODES])
    cols_p = jnp.concatenate([col, DUMMY + pad_iota % (ACC_ROWS - N_NODES)])
    rows3 = rows_p.reshape(NW, CHUNKS_PER_TILE, CHUNK)
    cols3 = cols_p.reshape(NW, CHUNKS_PER_TILE, CHUNK)

    cnt = _sc_degree(cols_p)
    xs, dinv = _tc_scale(input_rep, W_gcn, cnt)
    partials = _sc_edges(rows3, cols3, xs)
    hidden, score = _tc_final(partials, xs, dinv, b_gcn,
                              batch_idx.astype(jnp.int32), W_fc, b_fc,
                              bn_gamma, bn_beta)
    return (hidden, score)
